# baseline (speedup 1.0000x reference)
import jax
import jax.numpy as jnp
from jax.experimental import pallas as pl
from jax.experimental.pallas import tpu as pltpu


def _matmul_kernel(ci_ref, oh_ref, out_ref):
    k = pl.program_id(0)
    acc = jax.lax.dot_general(
        ci_ref[...], oh_ref[...],
        dimension_numbers=(((0,), (0,)), ((), ())),
        preferred_element_type=jnp.float32,
    )

    @pl.when(k == 0)
    def _init():
        out_ref[...] = acc

    @pl.when(k > 0)
    def _acc():
        out_ref[...] += acc


def _utility_kernel(xu_ref, xi_ref, cu_ref, cuser_ref, cb_ref, av_ref, out_ref):
    v = xu_ref[...] * cu_ref[...] + xi_ref[...] * cuser_ref[...][None, :, :]
    s = v.sum(axis=1) + cb_ref[...][:, :, 0]
    out_ref[...] = jnp.where(av_ref[...], s, jnp.float32(-1e20))


def kernel(x_u, x_i, user_onehot, availability, coef_u, coef_i, coef_intercept):
    batch, num_items, p_u = x_u.shape
    p_i = x_i.shape[2]
    num_users = user_onehot.shape[1]

    oh_t = user_onehot.T
    xu_t = x_u.transpose(1, 2, 0)
    xi_t = x_i.transpose(1, 2, 0)
    av_t = availability.T
    cu3 = coef_u[:, :, None]
    cb3 = coef_intercept[:, :, None]

    u_tile = 2000
    nk = num_users // u_tile
    coef_user_t = pl.pallas_call(
        _matmul_kernel,
        grid=(nk,),
        in_specs=[
            pl.BlockSpec((u_tile, p_i), lambda k: (k, 0)),
            pl.BlockSpec((u_tile, batch), lambda k: (k, 0)),
        ],
        out_specs=pl.BlockSpec((p_i, batch), lambda k: (0, 0)),
        out_shape=jax.ShapeDtypeStruct((p_i, batch), jnp.float32),
        compiler_params=pltpu.CompilerParams(
            dimension_semantics=("arbitrary",),
        ),
    )(coef_i, oh_t)

    n_tile = 40
    nn = num_items // n_tile
    out_t = pl.pallas_call(
        _utility_kernel,
        grid=(nn,),
        in_specs=[
            pl.BlockSpec((n_tile, p_u, batch), lambda i: (i, 0, 0)),
            pl.BlockSpec((n_tile, p_i, batch), lambda i: (i, 0, 0)),
            pl.BlockSpec((n_tile, p_u, 1), lambda i: (i, 0, 0)),
            pl.BlockSpec((p_i, batch), lambda i: (0, 0)),
            pl.BlockSpec((n_tile, 1, 1), lambda i: (i, 0, 0)),
            pl.BlockSpec((n_tile, batch), lambda i: (i, 0)),
        ],
        out_specs=pl.BlockSpec((n_tile, batch), lambda i: (i, 0)),
        out_shape=jax.ShapeDtypeStruct((num_items, batch), jnp.float32),
        compiler_params=pltpu.CompilerParams(
            dimension_semantics=("parallel",),
        ),
    )(xu_t, xi_t, cu3, coef_user_t, cb3, av_t)
    return out_t.T


# speedup vs baseline: 1.0515x; 1.0515x over previous
import jax
import jax.numpy as jnp
from jax.experimental import pallas as pl
from jax.experimental.pallas import tpu as pltpu


def _matmul_kernel(ci_ref, oh_ref, out_ref):
    k = pl.program_id(0)
    acc = jax.lax.dot_general(
        ci_ref[0], oh_ref[...],
        dimension_numbers=(((1,), (0,)), ((), ())),
        preferred_element_type=jnp.float32,
    )

    @pl.when(k == 0)
    def _init():
        out_ref[...] = acc

    @pl.when(k > 0)
    def _acc():
        out_ref[...] += acc


def _utility_kernel(xu_ref, xi_ref, cu_ref, cuser_ref, cb_ref, av_ref, out_ref):
    v = xu_ref[...] * cu_ref[...] + xi_ref[...] * cuser_ref[...][None, :, :]
    s = v.sum(axis=1) + cb_ref[...][:, :, 0]
    out_ref[...] = jnp.where(av_ref[...], s, jnp.float32(-1e20))


def kernel(x_u, x_i, user_onehot, availability, coef_u, coef_i, coef_intercept):
    batch, num_items, p_u = x_u.shape
    p_i = x_i.shape[2]
    num_users = user_onehot.shape[1]

    oh_t = user_onehot.T
    xu_t = x_u.transpose(1, 2, 0)
    xi_t = x_i.transpose(1, 2, 0)
    av_t = availability.T
    cu3 = coef_u[:, :, None]
    cb3 = coef_intercept[:, :, None]

    u_tile = 4000
    nk = num_users // u_tile
    ci_chunks = coef_i.T.reshape(p_i, nk, u_tile).transpose(1, 0, 2)
    coef_user_t = pl.pallas_call(
        _matmul_kernel,
        grid=(nk,),
        in_specs=[
            pl.BlockSpec((1, p_i, u_tile), lambda k: (k, 0, 0)),
            pl.BlockSpec((u_tile, batch), lambda k: (k, 0)),
        ],
        out_specs=pl.BlockSpec((p_i, batch), lambda k: (0, 0)),
        out_shape=jax.ShapeDtypeStruct((p_i, batch), jnp.float32),
        compiler_params=pltpu.CompilerParams(
            dimension_semantics=("arbitrary",),
        ),
    )(ci_chunks, oh_t)

    n_tile = 40
    nn = num_items // n_tile
    out_t = pl.pallas_call(
        _utility_kernel,
        grid=(nn,),
        in_specs=[
            pl.BlockSpec((n_tile, p_u, batch), lambda i: (i, 0, 0)),
            pl.BlockSpec((n_tile, p_i, batch), lambda i: (i, 0, 0)),
            pl.BlockSpec((n_tile, p_u, 1), lambda i: (i, 0, 0)),
            pl.BlockSpec((p_i, batch), lambda i: (0, 0)),
            pl.BlockSpec((n_tile, 1, 1), lambda i: (i, 0, 0)),
            pl.BlockSpec((n_tile, batch), lambda i: (i, 0)),
        ],
        out_specs=pl.BlockSpec((n_tile, batch), lambda i: (i, 0)),
        out_shape=jax.ShapeDtypeStruct((num_items, batch), jnp.float32),
        compiler_params=pltpu.CompilerParams(
            dimension_semantics=("parallel",),
        ),
    )(xu_t, xi_t, cu3, coef_user_t, cb3, av_t)
    return out_t.T
